# baseline (speedup 1.0000x reference)
import jax
import jax.numpy as jnp
from jax import lax
from jax.experimental import pallas as pl
from jax.experimental.pallas import tpu as pltpu

_TAU = 1.0
_NL = 4
_H = 224
_W = 224
_C = 96
_CP = 128
_BH = 32
_NBLK = _H // _BH

_TAPS = [(dh, dw) for dh in range(3) for dw in range(3)]
_GROUPS = [_TAPS[0:2], _TAPS[2:4], _TAPS[4:6], _TAPS[6:8], _TAPS[8:9]]


def _prep_body(ch8_ref, g8_ref, chT_ref, gT_ref, w1_ref, w2_ref, w3_ref,
               cm_ref, wsc_ref):
    f32 = jnp.float32
    bf16 = jnp.bfloat16
    inv_tau = 1.0 / _TAU
    l8 = (ch8_ref[:, :] + g8_ref[:, :]) * inv_tau
    lT = (chT_ref[:, :] + gT_ref[:, :]) * inv_tau

    rows = []
    cmdT, cmsT = [], []
    for i in range(_NL):
        a = l8[2 * i:2 * i + 1, :]
        b = l8[2 * i + 1:2 * i + 2, :]
        m = jnp.maximum(a, b)
        ea = jnp.exp(a - m)
        eb = jnp.exp(b - m)
        s = ea + eb
        rows.append(ea / s)
        rows.append(eb / s)
        aT = lT[:, 2 * i:2 * i + 1]
        bT = lT[:, 2 * i + 1:2 * i + 2]
        mT = jnp.maximum(aT, bT)
        eaT = jnp.exp(aT - mT)
        ebT = jnp.exp(bT - mT)
        sT = eaT + ebT
        cmdT.append(eaT / sT)
        cmsT.append(ebT / sT)
    cm_ref[:, :] = jnp.concatenate(rows, axis=0)

    wrefs = [w1_ref, w2_ref, w3_ref]
    for li in range(3):
        cd_p, cs_p = cmdT[li], cmsT[li]
        for dh in range(3):
            blocks = [jnp.concatenate([wrefs[li][dh, dw, :, :] * cd_p,
                                       wrefs[li][dh, dw, :, :] * cs_p], axis=1)
                      for dw in range(3)]
            wsc_ref[li, dh, :, :] = jnp.concatenate(blocks, axis=0).astype(bf16)


def _smb_body(fea_ref, spa_ref, cm_ref, w0g_ref, wsc_ref, wc_ref, bc_ref,
              y_ref, s0_ref, s1_ref):
    pid = pl.program_id(0)
    g0 = pid * _BH
    f32 = jnp.float32
    bf16 = jnp.bfloat16

    cmrow = cm_ref[:, :]

    zstrip = jnp.zeros((_BH + 6, 1, _CP), dtype=bf16)
    s0_ref[:, 0:1, 0:_CP] = zstrip
    s0_ref[:, _W - 1:_W, 2 * _CP:3 * _CP] = zstrip
    s1_ref[:, 0:1, 0:_CP] = zstrip
    s1_ref[:, _W - 1:_W, 2 * _CP:3 * _CP] = zstrip

    def store_stacked(sref, nrows, x_bf):
        sref[0:nrows, 1:_W, 0:_CP] = x_bf[:, 0:_W - 1]
        sref[0:nrows, 0:_W, _CP:2 * _CP] = x_bf
        sref[0:nrows, 0:_W - 1, 2 * _CP:3 * _CP] = x_bf[:, 1:_W]

    spaw = spa_ref[pl.ds(g0, _BH + 8), :]

    def mults(layer, nrows):
        spal = spaw[layer:layer + nrows][:, :, None]
        ridx = lax.broadcasted_iota(jnp.int32, (nrows, _W), 0) + (g0 - 4 + layer)
        valid = ((ridx >= 0) & (ridx < _H)).astype(f32)[:, :, None]
        cd = cmrow[2 * layer - 2:2 * layer - 1, :].reshape(1, 1, _CP)
        cs = cmrow[2 * layer - 1:2 * layer, :].reshape(1, 1, _CP)
        return spal, spal * cs + valid * cd

    centers = []

    n1 = _BH + 6
    acc = None
    for gi, grp in enumerate(_GROUPS):
        xs = jnp.concatenate(
            [fea_ref[pl.ds(g0 + dh, n1), pl.ds(dw, _W), :] for dh, dw in grp],
            axis=2).reshape(n1 * _W, len(grp) * _CP)
        wg = w0g_ref[gi, 0:len(grp) * _CP, :]
        d = jnp.dot(xs, wg, preferred_element_type=f32)
        acc = d if acc is None else acc + d
    spal, A = mults(1, n1)
    x = jnp.maximum(acc.reshape(n1, _W, _CP) * A, 0.0)
    store_stacked(s0_ref, n1, x.astype(bf16))
    centers.append(x[3:3 + _BH].reshape(_BH * _W, _CP).astype(bf16))

    for layer in range(2, _NL + 1):
        nout = _BH + 8 - 2 * layer
        sprev = s0_ref if layer % 2 == 0 else s1_ref
        scur = s1_ref if layer % 2 == 0 else s0_ref
        acc = None
        for dh in range(3):
            xs = sprev[dh:dh + nout, :, :].reshape(nout * _W, 3 * _CP)
            wg = wsc_ref[layer - 2, dh, :, :]
            d = jnp.dot(xs, wg, preferred_element_type=f32)
            acc = d if acc is None else acc + d
        fd = acc[:, :_CP].reshape(nout, _W, _CP)
        fs = acc[:, _CP:].reshape(nout, _W, _CP)
        spal, A = mults(layer, nout)
        x = jnp.maximum(fd * A + fs * spal, 0.0)
        if layer < _NL:
            store_stacked(scur, nout, x.astype(bf16))
        centers.append(
            x[4 - layer:4 - layer + _BH].reshape(_BH * _W, _CP).astype(bf16))

    xcat = jnp.concatenate(centers, axis=1)
    y = jnp.dot(xcat, wc_ref[:, :], preferred_element_type=f32) + bc_ref[:, :]
    y_ref[:, :, :] = y[:, :_C].reshape(_BH, _W, _C)


def kernel(fea, spa_mask, ch_mask, W0, W1, W2, W3, Wc, bc):
    f32 = jnp.float32
    bf16 = jnp.bfloat16
    pc = _CP - _C

    u = jax.random.uniform(jax.random.key(1234), ch_mask.shape,
                           minval=1e-8, maxval=1.0 - 1e-8, dtype=ch_mask.dtype)
    g = -jnp.log(-jnp.log(u))
    ch8 = jnp.pad(ch_mask.reshape(2 * _NL, _C), ((0, 0), (0, pc)))
    g8 = jnp.pad(g.reshape(2 * _NL, _C), ((0, 0), (0, pc)))
    chT = ch8.T
    gT = g8.T
    w1 = jnp.pad(W1, ((0, 0), (0, 0), (0, pc), (0, pc)))
    w2 = jnp.pad(W2, ((0, 0), (0, 0), (0, pc), (0, pc)))
    w3 = jnp.pad(W3, ((0, 0), (0, 0), (0, pc), (0, pc)))

    cm8, wsc = pl.pallas_call(
        _prep_body,
        in_specs=[
            pl.BlockSpec((2 * _NL, _CP), lambda: (0, 0)),
            pl.BlockSpec((2 * _NL, _CP), lambda: (0, 0)),
            pl.BlockSpec((_CP, 2 * _NL), lambda: (0, 0)),
            pl.BlockSpec((_CP, 2 * _NL), lambda: (0, 0)),
            pl.BlockSpec((3, 3, _CP, _CP), lambda: (0, 0, 0, 0)),
            pl.BlockSpec((3, 3, _CP, _CP), lambda: (0, 0, 0, 0)),
            pl.BlockSpec((3, 3, _CP, _CP), lambda: (0, 0, 0, 0)),
        ],
        out_specs=[
            pl.BlockSpec((2 * _NL, _CP), lambda: (0, 0)),
            pl.BlockSpec((3, 3, 3 * _CP, 2 * _CP), lambda: (0, 0, 0, 0)),
        ],
        out_shape=[
            jax.ShapeDtypeStruct((2 * _NL, _CP), f32),
            jax.ShapeDtypeStruct((3, 3, 3 * _CP, 2 * _CP), bf16),
        ],
    )(ch8, g8, chT, gT, w1, w2, w3)

    w0p = jnp.pad(W0, ((0, 0), (0, 0), (0, pc), (0, pc))).astype(bf16)
    w0g = jnp.stack(
        [jnp.concatenate(
            [w0p[dh, dw] for dh, dw in grp]
            + ([jnp.zeros((_CP, _CP), dtype=bf16)] if len(grp) == 1 else []),
            axis=0)
         for grp in _GROUPS], axis=0)

    fea_p = jnp.pad(fea[0], ((4, 4), (1, 1), (0, pc))).astype(bf16)
    spa_p = jnp.pad(spa_mask[0, :, :, 0], ((4, 4), (0, 0)))
    wc = jnp.pad(Wc.reshape(_NL, _C, _C),
                 ((0, 0), (0, pc), (0, pc))).astype(bf16).reshape(_NL * _CP, _CP)
    bcr = jnp.pad(bc.reshape(1, _C), ((0, 0), (0, pc)))

    y = pl.pallas_call(
        _smb_body,
        grid=(_NBLK,),
        in_specs=[
            pl.BlockSpec(fea_p.shape, lambda i: (0, 0, 0)),
            pl.BlockSpec(spa_p.shape, lambda i: (0, 0)),
            pl.BlockSpec((2 * _NL, _CP), lambda i: (0, 0)),
            pl.BlockSpec((5, 2 * _CP, _CP), lambda i: (0, 0, 0)),
            pl.BlockSpec((3, 3, 3 * _CP, 2 * _CP), lambda i: (0, 0, 0, 0)),
            pl.BlockSpec((_NL * _CP, _CP), lambda i: (0, 0)),
            pl.BlockSpec((1, _CP), lambda i: (0, 0)),
        ],
        out_specs=pl.BlockSpec((_BH, _W, _C), lambda i: (i, 0, 0)),
        out_shape=jax.ShapeDtypeStruct((_H, _W, _C), f32),
        scratch_shapes=[
            pltpu.VMEM((_BH + 6, _W, 3 * _CP), bf16),
            pltpu.VMEM((_BH + 6, _W, 3 * _CP), bf16),
        ],
        compiler_params=pltpu.CompilerParams(
            dimension_semantics=("arbitrary",)),
    )(fea_p, spa_p, cm8, w0g, wsc, wc, bcr)

    return y.reshape(1, _H, _W, _C), cm8[:, :_C].reshape(1, _NL, 2, _C)


# speedup vs baseline: 1.4031x; 1.4031x over previous
import jax
import jax.numpy as jnp
from jax import lax
from jax.experimental import pallas as pl
from jax.experimental.pallas import tpu as pltpu

_TAU = 1.0
_NL = 4
_H = 224
_W = 224
_C = 96
_CP = 128
_BH = 32
_NBLK = _H // _BH


def _smb_body(fea_ref, spa_ref, ch8_ref, g8_ref, chT_ref, gT_ref,
              w0_ref, w1_ref, w2_ref, w3_ref, wc_ref, bc_ref,
              y_ref, cm_ref, s0_ref, s1_ref):
    pid = pl.program_id(0)
    g0 = pid * _BH
    f32 = jnp.float32
    bf16 = jnp.bfloat16

    inv_tau = 1.0 / _TAU
    l8 = (ch8_ref[:, :] + g8_ref[:, :]) * inv_tau
    lT = (chT_ref[:, :] + gT_ref[:, :]) * inv_tau

    cmd_r, cms_r, cmdT, cmsT = [], [], [], []
    for i in range(_NL):
        a = l8[2 * i:2 * i + 1, :]
        b = l8[2 * i + 1:2 * i + 2, :]
        m = jnp.maximum(a, b)
        ea = jnp.exp(a - m)
        eb = jnp.exp(b - m)
        s = ea + eb
        cmd_r.append(ea / s)
        cms_r.append(eb / s)
        aT = lT[:, 2 * i:2 * i + 1]
        bT = lT[:, 2 * i + 1:2 * i + 2]
        mT = jnp.maximum(aT, bT)
        eaT = jnp.exp(aT - mT)
        ebT = jnp.exp(bT - mT)
        sT = eaT + ebT
        cmdT.append(eaT / sT)
        cmsT.append(ebT / sT)

    rows = []
    for i in range(_NL):
        rows.append(cmd_r[i])
        rows.append(cms_r[i])
    cm_ref[0, :, :] = jnp.concatenate(rows, axis=0)

    zcol = jnp.zeros((_BH + 6, 1, _CP), dtype=bf16)
    s0_ref[:, 0:1, :] = zcol
    s0_ref[:, _W + 1:_W + 2, :] = zcol
    s1_ref[:, 0:1, :] = zcol
    s1_ref[:, _W + 1:_W + 2, :] = zcol

    spaw = spa_ref[pl.ds(g0, _BH + 8), :]

    def mults(layer, nrows):
        spal = spaw[layer:layer + nrows][:, :, None]
        ridx = lax.broadcasted_iota(jnp.int32, (nrows, _W), 0) + (g0 - 4 + layer)
        valid = ((ridx >= 0) & (ridx < _H)).astype(f32)[:, :, None]
        cs = cms_r[layer - 1].reshape(1, 1, _CP)
        cd = cmd_r[layer - 1].reshape(1, 1, _CP)
        return spal, spal * cs + valid * cd

    centers = []

    taps = [(dh, dw) for dh in range(3) for dw in range(3)]
    groups = [taps[0:2], taps[2:4], taps[4:6], taps[6:8], taps[8:9]]

    n1 = _BH + 6
    xw = [fea_ref[pl.ds(g0, n1 + 2), pl.ds(dw, _W), :] for dw in range(3)]
    acc = None
    for grp in groups:
        wg = jnp.concatenate(
            [w0_ref[dh, dw, :, :] for dh, dw in grp], axis=0)
        xs = jnp.concatenate(
            [xw[dw][dh:dh + n1] for dh, dw in grp], axis=2)
        d = jnp.dot(xs.reshape(n1 * _W, len(grp) * _CP), wg,
                    preferred_element_type=f32)
        acc = d if acc is None else acc + d
    spal, A = mults(1, n1)
    x = jnp.maximum(acc.reshape(n1, _W, _CP) * A, 0.0)
    s0_ref[0:n1, 1:_W + 1, :] = x.astype(bf16)
    centers.append(x[3:3 + _BH].reshape(_BH * _W, _CP).astype(bf16))

    wrefs = {2: w1_ref, 3: w2_ref, 4: w3_ref}
    for layer in range(2, _NL + 1):
        nout = _BH + 8 - 2 * layer
        sprev = s0_ref if layer % 2 == 0 else s1_ref
        scur = s1_ref if layer % 2 == 0 else s0_ref
        wref = wrefs[layer]
        cd_p = cmdT[layer - 2]
        cs_p = cmsT[layer - 2]
        xw = [sprev[0:nout + 2, pl.ds(dw, _W), :] for dw in range(3)]
        acc = None
        for grp in groups:
            wg = jnp.concatenate(
                [jnp.concatenate([wref[dh, dw, :, :] * cd_p,
                                  wref[dh, dw, :, :] * cs_p], axis=1)
                 for dh, dw in grp], axis=0).astype(bf16)
            xs = jnp.concatenate(
                [xw[dw][dh:dh + nout] for dh, dw in grp], axis=2)
            d = jnp.dot(xs.reshape(nout * _W, len(grp) * _CP), wg,
                        preferred_element_type=f32)
            acc = d if acc is None else acc + d
        fd = acc[:, :_CP].reshape(nout, _W, _CP)
        fs = acc[:, _CP:].reshape(nout, _W, _CP)
        spal, A = mults(layer, nout)
        x = jnp.maximum(fd * A + fs * spal, 0.0)
        if layer < _NL:
            scur[0:nout, 1:_W + 1, :] = x.astype(bf16)
        centers.append(
            x[4 - layer:4 - layer + _BH].reshape(_BH * _W, _CP).astype(bf16))

    xcat = jnp.concatenate(centers, axis=1)
    y = jnp.dot(xcat, wc_ref[:, :], preferred_element_type=f32) + bc_ref[:, :]
    y_ref[:, :, :] = y[:, :_C].reshape(_BH, _W, _C)


def kernel(fea, spa_mask, ch_mask, W0, W1, W2, W3, Wc, bc):
    f32 = jnp.float32
    bf16 = jnp.bfloat16
    pc = _CP - _C

    u = jax.random.uniform(jax.random.key(1234), ch_mask.shape,
                           minval=1e-8, maxval=1.0 - 1e-8, dtype=ch_mask.dtype)
    g = -jnp.log(-jnp.log(u))
    ch8 = jnp.pad(ch_mask.reshape(2 * _NL, _C), ((0, 0), (0, pc)))
    g8 = jnp.pad(g.reshape(2 * _NL, _C), ((0, 0), (0, pc)))
    chT = ch8.T
    gT = g8.T

    fea_p = jnp.pad(fea[0], ((4, 4), (1, 1), (0, pc))).astype(bf16)
    spa_p = jnp.pad(spa_mask[0, :, :, 0], ((4, 4), (0, 0)))
    w0 = jnp.pad(W0, ((0, 0), (0, 0), (0, pc), (0, pc))).astype(bf16)
    w1 = jnp.pad(W1, ((0, 0), (0, 0), (0, pc), (0, pc)))
    w2 = jnp.pad(W2, ((0, 0), (0, 0), (0, pc), (0, pc)))
    w3 = jnp.pad(W3, ((0, 0), (0, 0), (0, pc), (0, pc)))
    wc = jnp.pad(Wc.reshape(_NL, _C, _C),
                 ((0, 0), (0, pc), (0, pc))).astype(bf16).reshape(_NL * _CP, _CP)
    bcr = jnp.pad(bc.reshape(1, _C), ((0, 0), (0, pc)))

    y, cm8 = pl.pallas_call(
        _smb_body,
        grid=(_NBLK,),
        in_specs=[
            pl.BlockSpec(fea_p.shape, lambda i: (0, 0, 0)),
            pl.BlockSpec(spa_p.shape, lambda i: (0, 0)),
            pl.BlockSpec((2 * _NL, _CP), lambda i: (0, 0)),
            pl.BlockSpec((2 * _NL, _CP), lambda i: (0, 0)),
            pl.BlockSpec((_CP, 2 * _NL), lambda i: (0, 0)),
            pl.BlockSpec((_CP, 2 * _NL), lambda i: (0, 0)),
            pl.BlockSpec((3, 3, _CP, _CP), lambda i: (0, 0, 0, 0)),
            pl.BlockSpec((3, 3, _CP, _CP), lambda i: (0, 0, 0, 0)),
            pl.BlockSpec((3, 3, _CP, _CP), lambda i: (0, 0, 0, 0)),
            pl.BlockSpec((3, 3, _CP, _CP), lambda i: (0, 0, 0, 0)),
            pl.BlockSpec((_NL * _CP, _CP), lambda i: (0, 0)),
            pl.BlockSpec((1, _CP), lambda i: (0, 0)),
        ],
        out_specs=[
            pl.BlockSpec((_BH, _W, _C), lambda i: (i, 0, 0)),
            pl.BlockSpec((1, 2 * _NL, _CP), lambda i: (i, 0, 0)),
        ],
        out_shape=[
            jax.ShapeDtypeStruct((_H, _W, _C), f32),
            jax.ShapeDtypeStruct((_NBLK, 2 * _NL, _CP), f32),
        ],
        scratch_shapes=[
            pltpu.VMEM((_BH + 6, _W + 2, _CP), bf16),
            pltpu.VMEM((_BH + 6, _W + 2, _CP), bf16),
        ],
        compiler_params=pltpu.CompilerParams(
            dimension_semantics=("arbitrary",)),
    )(fea_p, spa_p, ch8, g8, chT, gT, w0, w1, w2, w3, wc, bcr)

    return y.reshape(1, _H, _W, _C), cm8[0, :, :_C].reshape(1, _NL, 2, _C)
